# baseline (speedup 1.0000x reference)
import jax
import jax.numpy as jnp
from jax.experimental import pallas as pl
from jax.experimental.pallas import tpu as pltpu

_IN_FEATURES = 28 * 28
_OUT_FEATURES = 5
_TAU = 2.0
_V_THRESHOLD = 0.8
_M_PAD = 8

_BM = 1024


def _vpr_block_kernel(x_ref, w_ref, cur_ref, spk_ref):
    x3 = x_ref[...]
    x = x3.reshape(x3.shape[0], _IN_FEATURES)
    w = w_ref[...]
    cur = jax.lax.dot_general(
        w, x,
        dimension_numbers=(((1,), (1,)), ((), ())),
        preferred_element_type=jnp.float32,
    )
    cur_ref[...] = cur
    spk_ref[...] = (cur / _TAU >= _V_THRESHOLD).astype(jnp.float32)


@jax.jit
def kernel(xb, w_pad):
    b = xb.shape[0]
    bm = _BM if b >= _BM else max(8, b)
    nb = pl.cdiv(b, bm)
    b_pad = nb * bm
    if b_pad != b:
        xb = jnp.pad(xb, ((0, b_pad - b), (0, 0), (0, 0)))

    cur_t, spk_t = pl.pallas_call(
        _vpr_block_kernel,
        out_shape=(
            jax.ShapeDtypeStruct((_M_PAD, b_pad), jnp.float32),
            jax.ShapeDtypeStruct((_M_PAD, b_pad), jnp.float32),
        ),
        grid=(nb,),
        in_specs=[
            pl.BlockSpec((bm, 28, 28), lambda i: (i, 0, 0)),
            pl.BlockSpec((_M_PAD, _IN_FEATURES), lambda i: (0, 0)),
        ],
        out_specs=(
            pl.BlockSpec((_M_PAD, bm), lambda i: (0, i)),
            pl.BlockSpec((_M_PAD, bm), lambda i: (0, i)),
        ),
        compiler_params=pltpu.CompilerParams(
            dimension_semantics=("parallel",)),
    )(xb.astype(jnp.float32), w_pad.astype(jnp.float32))

    cur = cur_t[:_OUT_FEATURES, :b].T
    spikes = spk_t[:_OUT_FEATURES, :b].T
    return spikes, cur


# speedup vs baseline: 4.8491x; 4.8491x over previous
import jax
import jax.numpy as jnp
from jax.experimental import pallas as pl
from jax.experimental.pallas import tpu as pltpu

_IN_FEATURES = 28 * 28
_OUT_FEATURES = 5
_TAU = 2.0
_V_THRESHOLD = 0.8
_M_PAD = 8

_BM = 2048


def _vpr_block_kernel(xt_ref, w_ref, cur_ref, spk_ref):
    xt = xt_ref[...]
    w = w_ref[...]
    cur = jax.lax.dot_general(
        w, xt,
        dimension_numbers=(((1,), (0,)), ((), ())),
        preferred_element_type=jnp.float32,
    )
    cur_ref[...] = cur
    spk_ref[...] = (cur / _TAU >= _V_THRESHOLD).astype(jnp.float32)


@jax.jit
def kernel(xb, w_pad):
    b = xb.shape[0]
    xt = jnp.reshape(xb, (b, _IN_FEATURES)).astype(jnp.float32).T

    bm = _BM if b >= _BM else max(128, b)
    nb = pl.cdiv(b, bm)
    b_pad = nb * bm
    if b_pad != b:
        xt = jnp.pad(xt, ((0, 0), (0, b_pad - b)))

    cur_t, spk_t = pl.pallas_call(
        _vpr_block_kernel,
        out_shape=(
            jax.ShapeDtypeStruct((_M_PAD, b_pad), jnp.float32),
            jax.ShapeDtypeStruct((_M_PAD, b_pad), jnp.float32),
        ),
        grid=(nb,),
        in_specs=[
            pl.BlockSpec((_IN_FEATURES, bm), lambda i: (0, i)),
            pl.BlockSpec((_M_PAD, _IN_FEATURES), lambda i: (0, 0)),
        ],
        out_specs=(
            pl.BlockSpec((_M_PAD, bm), lambda i: (0, i)),
            pl.BlockSpec((_M_PAD, bm), lambda i: (0, i)),
        ),
        compiler_params=pltpu.CompilerParams(
            dimension_semantics=("parallel",)),
    )(xt, w_pad.astype(jnp.float32))

    cur = cur_t[:_OUT_FEATURES, :b].T
    spikes = spk_t[:_OUT_FEATURES, :b].T
    return spikes, cur
